# Initial kernel scaffold; baseline (speedup 1.0000x reference)
import functools

import jax
import jax.numpy as jnp
from jax import lax
from jax.experimental import pallas as pl
from jax.experimental.pallas import tpu as pltpu
from jax.experimental.pallas import tpu_sc as plsc

_NC = 2
_NS = 16
_NW = _NC * _NS
_L = 16
_TBL = 128


def _tile_body(species_hbm, conv_hbm, out_hbm, spec_v, conv_v, out_v):
    n = spec_v.shape[0]
    wid = lax.axis_index("s") * _NC + lax.axis_index("c")
    base = wid * n
    pltpu.sync_copy(conv_hbm, conv_v)
    pltpu.sync_copy(species_hbm.at[pl.ds(base, n)], spec_v)

    def step(i, carry):
        off = i * _L
        idx = spec_v[pl.ds(off, _L)]
        out_v[pl.ds(off, _L)] = plsc.load_gather(conv_v, [idx])
        return carry

    lax.fori_loop(0, n // _L, step, 0, unroll=8)
    pltpu.sync_copy(out_v, out_hbm.at[pl.ds(base, n)])


@functools.lru_cache(maxsize=None)
def _make_gather(n_total: int):
    n_per = n_total // _NW
    mesh = plsc.VectorSubcoreMesh(core_axis_name="c", subcore_axis_name="s")
    return pl.kernel(
        _tile_body,
        out_type=jax.ShapeDtypeStruct((n_total,), jnp.int32),
        mesh=mesh,
        scratch_types=[
            pltpu.VMEM((n_per,), jnp.int32),
            pltpu.VMEM((_TBL,), jnp.int32),
            pltpu.VMEM((n_per,), jnp.int32),
        ],
    )


def kernel(species, coordinates, conv_tensor):
    b, a = species.shape
    flat = species.reshape(-1).astype(jnp.int32)
    conv_pad = jnp.zeros((_TBL,), jnp.int32).at[: conv_tensor.shape[0]].set(
        conv_tensor.astype(jnp.int32)
    )
    converted = _make_gather(flat.shape[0])(flat, conv_pad)
    return converted.reshape(b, a).astype(conv_tensor.dtype), coordinates


# speedup vs baseline: 154.6533x; 154.6533x over previous
import functools

import jax
import jax.numpy as jnp
from jax import lax
from jax.experimental import pallas as pl
from jax.experimental.pallas import tpu as pltpu
from jax.experimental.pallas import tpu_sc as plsc

_NC = 2
_NS = 16
_NW = _NC * _NS
_L = 16
_TBL = 128


def _tile_body(species_hbm, conv_hbm, out_hbm, spec_v, conv_v, out_v):
    n = spec_v.shape[0]
    wid = lax.axis_index("s") * _NC + lax.axis_index("c")
    base = wid * n
    pltpu.sync_copy(conv_hbm, conv_v)
    pltpu.sync_copy(species_hbm.at[pl.ds(base, n)], spec_v)

    def step(i, carry):
        off = i * _L
        idx = spec_v[pl.ds(off, _L)]
        out_v[pl.ds(off, _L)] = plsc.load_gather(conv_v, [idx])
        return carry

    lax.fori_loop(0, n // _L, step, 0, unroll=8)
    pltpu.sync_copy(out_v, out_hbm.at[pl.ds(base, n)])


@functools.lru_cache(maxsize=None)
def _make_gather(n_total: int):
    n_per = n_total // _NW
    mesh = plsc.VectorSubcoreMesh(core_axis_name="c", subcore_axis_name="s")
    return pl.kernel(
        _tile_body,
        out_type=jax.ShapeDtypeStruct((n_total,), jnp.int32),
        mesh=mesh,
        scratch_types=[
            pltpu.VMEM((n_per,), jnp.int32),
            pltpu.VMEM((_TBL,), jnp.int32),
            pltpu.VMEM((n_per,), jnp.int32),
        ],
        compiler_params=pltpu.CompilerParams(needs_layout_passes=False),
    )


def kernel(species, coordinates, conv_tensor):
    b, a = species.shape
    flat = species.reshape(-1).astype(jnp.int32)
    conv_pad = jnp.zeros((_TBL,), jnp.int32).at[: conv_tensor.shape[0]].set(
        conv_tensor.astype(jnp.int32)
    )
    converted = _make_gather(flat.shape[0])(flat, conv_pad)
    return converted.reshape(b, a).astype(conv_tensor.dtype), coordinates


# speedup vs baseline: 186.1505x; 1.2037x over previous
import functools

import jax
import jax.numpy as jnp
from jax import lax
from jax.experimental import pallas as pl
from jax.experimental.pallas import tpu as pltpu
from jax.experimental.pallas import tpu_sc as plsc

_NC = 2
_NS = 16
_NW = _NC * _NS
_L = 16
_TBL = 128


def _tile_body(species_hbm, conv_hbm, out_hbm, spec_v, conv_v, out_v):
    n = spec_v.shape[0]
    wid = lax.axis_index("s") * _NC + lax.axis_index("c")
    base = wid * n
    pltpu.sync_copy(conv_hbm, conv_v)
    pltpu.sync_copy(species_hbm.at[pl.ds(base, n)], spec_v)

    @plsc.parallel_loop(0, n // _L, unroll=8)
    def _(i):
        off = i * _L
        idx = spec_v[pl.ds(off, _L)]
        out_v[pl.ds(off, _L)] = plsc.load_gather(conv_v, [idx])
    pltpu.sync_copy(out_v, out_hbm.at[pl.ds(base, n)])


@functools.lru_cache(maxsize=None)
def _make_gather(n_total: int):
    n_per = n_total // _NW
    mesh = plsc.VectorSubcoreMesh(core_axis_name="c", subcore_axis_name="s")
    return pl.kernel(
        _tile_body,
        out_type=jax.ShapeDtypeStruct((n_total,), jnp.int32),
        mesh=mesh,
        scratch_types=[
            pltpu.VMEM((n_per,), jnp.int32),
            pltpu.VMEM((_TBL,), jnp.int32),
            pltpu.VMEM((n_per,), jnp.int32),
        ],
        compiler_params=pltpu.CompilerParams(needs_layout_passes=False),
    )


def kernel(species, coordinates, conv_tensor):
    b, a = species.shape
    flat = species.reshape(-1).astype(jnp.int32)
    conv_pad = jnp.zeros((_TBL,), jnp.int32).at[: conv_tensor.shape[0]].set(
        conv_tensor.astype(jnp.int32)
    )
    converted = _make_gather(flat.shape[0])(flat, conv_pad)
    return converted.reshape(b, a).astype(conv_tensor.dtype), coordinates


# speedup vs baseline: 187.1303x; 1.0053x over previous
import functools

import jax
import jax.numpy as jnp
from jax import lax
from jax.experimental import pallas as pl
from jax.experimental.pallas import tpu as pltpu
from jax.experimental.pallas import tpu_sc as plsc

_NC = 2
_NS = 16
_NW = _NC * _NS
_L = 16
_TBL = 128
_CHUNKS = 4


@functools.lru_cache(maxsize=None)
def _make_gather(n_total: int):
    n_per = n_total // _NW
    m = n_per // _CHUNKS
    mesh = plsc.VectorSubcoreMesh(core_axis_name="c", subcore_axis_name="s")

    def body(species_hbm, conv_hbm, out_hbm,
             conv_v, s0, s1, o0, o1, si0, si1, so0, so1):
        wid = lax.axis_index("s") * _NC + lax.axis_index("c")
        base = wid * n_per
        pltpu.sync_copy(conv_hbm, conv_v)
        sbuf, obuf = (s0, s1), (o0, o1)
        sin, sout = (si0, si1), (so0, so1)
        in_cp = [None, None]
        out_cp = [None, None]
        in_cp[0] = pltpu.async_copy(species_hbm.at[pl.ds(base, m)], s0, si0)
        for c in range(_CHUNKS):
            b = c & 1
            if c + 1 < _CHUNKS:
                in_cp[1 - b] = pltpu.async_copy(
                    species_hbm.at[pl.ds(base + (c + 1) * m, m)],
                    sbuf[1 - b], sin[1 - b])
            in_cp[b].wait()
            if c >= 2:
                out_cp[b].wait()

            @plsc.parallel_loop(0, m // _L, unroll=8)
            def _(i, spec_v=sbuf[b], out_v=obuf[b]):
                off = i * _L
                out_v[pl.ds(off, _L)] = plsc.load_gather(
                    conv_v, [spec_v[pl.ds(off, _L)]])

            out_cp[b] = pltpu.async_copy(
                obuf[b], out_hbm.at[pl.ds(base + c * m, m)], sout[b])
        out_cp[0].wait()
        out_cp[1].wait()

    return pl.kernel(
        body,
        out_type=jax.ShapeDtypeStruct((n_total,), jnp.int32),
        mesh=mesh,
        scratch_types=[
            pltpu.VMEM((_TBL,), jnp.int32),
            pltpu.VMEM((m,), jnp.int32),
            pltpu.VMEM((m,), jnp.int32),
            pltpu.VMEM((m,), jnp.int32),
            pltpu.VMEM((m,), jnp.int32),
            pltpu.SemaphoreType.DMA,
            pltpu.SemaphoreType.DMA,
            pltpu.SemaphoreType.DMA,
            pltpu.SemaphoreType.DMA,
        ],
        compiler_params=pltpu.CompilerParams(needs_layout_passes=False),
    )


def kernel(species, coordinates, conv_tensor):
    b, a = species.shape
    flat = species.reshape(-1).astype(jnp.int32)
    conv_pad = jnp.zeros((_TBL,), jnp.int32).at[: conv_tensor.shape[0]].set(
        conv_tensor.astype(jnp.int32)
    )
    converted = _make_gather(flat.shape[0])(flat, conv_pad)
    return converted.reshape(b, a).astype(conv_tensor.dtype), coordinates


# speedup vs baseline: 190.0791x; 1.0158x over previous
import functools

import jax
import jax.numpy as jnp
from jax import lax
from jax.experimental import pallas as pl
from jax.experimental.pallas import tpu as pltpu
from jax.experimental.pallas import tpu_sc as plsc

_NC = 2
_NS = 16
_NW = _NC * _NS
_L = 16
_TBL = 128
_NTBL = 120
_CHUNKS = 4


@functools.lru_cache(maxsize=None)
def _make_gather(n_total: int):
    n_per = n_total // _NW
    m = n_per // _CHUNKS
    mesh = plsc.VectorSubcoreMesh(core_axis_name="c", subcore_axis_name="s")

    def body(species_hbm, conv_hbm, out_hbm,
             conv_v, s0, s1, o0, o1, si0, si1, so0, so1):
        wid = lax.axis_index("s") * _NC + lax.axis_index("c")
        base = wid * n_per
        pltpu.sync_copy(conv_hbm, conv_v.at[pl.ds(0, _NTBL)])
        sbuf, obuf = (s0, s1), (o0, o1)
        sin, sout = (si0, si1), (so0, so1)
        in_cp = [None, None]
        out_cp = [None, None]
        in_cp[0] = pltpu.async_copy(species_hbm.at[pl.ds(base, m)], s0, si0)
        for c in range(_CHUNKS):
            b = c & 1
            if c + 1 < _CHUNKS:
                in_cp[1 - b] = pltpu.async_copy(
                    species_hbm.at[pl.ds(base + (c + 1) * m, m)],
                    sbuf[1 - b], sin[1 - b])
            in_cp[b].wait()
            if c >= 2:
                out_cp[b].wait()

            @plsc.parallel_loop(0, m // _L, unroll=8)
            def _(i, spec_v=sbuf[b], out_v=obuf[b]):
                off = i * _L
                out_v[pl.ds(off, _L)] = plsc.load_gather(
                    conv_v, [spec_v[pl.ds(off, _L)]])

            out_cp[b] = pltpu.async_copy(
                obuf[b], out_hbm.at[pl.ds(base + c * m, m)], sout[b])
        out_cp[0].wait()
        out_cp[1].wait()

    return pl.kernel(
        body,
        out_type=jax.ShapeDtypeStruct((n_total,), jnp.int32),
        mesh=mesh,
        scratch_types=[
            pltpu.VMEM((_TBL,), jnp.int32),
            pltpu.VMEM((m,), jnp.int32),
            pltpu.VMEM((m,), jnp.int32),
            pltpu.VMEM((m,), jnp.int32),
            pltpu.VMEM((m,), jnp.int32),
            pltpu.SemaphoreType.DMA,
            pltpu.SemaphoreType.DMA,
            pltpu.SemaphoreType.DMA,
            pltpu.SemaphoreType.DMA,
        ],
        compiler_params=pltpu.CompilerParams(needs_layout_passes=False),
    )


def kernel(species, coordinates, conv_tensor):
    b, a = species.shape
    flat = species.reshape(-1).astype(jnp.int32)
    converted = _make_gather(flat.shape[0])(flat, conv_tensor.astype(jnp.int32))
    return converted.reshape(b, a).astype(conv_tensor.dtype), coordinates
